# baseline (speedup 1.0000x reference)
import jax
import jax.numpy as jnp
from jax import lax
from jax.experimental import pallas as pl
from jax.experimental.pallas import tpu as pltpu
from jax.experimental.pallas import tpu_sc as plsc

M = 1000000
B = 16384
D = 64
L = 16
NC = 2
NS = 16
NW = NC * NS
MPAD = 1 << 20
RPT = MPAD // NW
JBITS = 14
SPW = B // NW
CH = 128
NCH = SPW // CH
INVALID = 0x7FFFFFFF


def _tag_body(idx_hbm, tag_hbm, idx_v, tag_v):
    cid = lax.axis_index("c")
    sid = lax.axis_index("s")
    wid = cid * NS + sid
    lanes = lax.iota(jnp.int32, L)
    shift = jnp.minimum(lanes + 1, L - 1)

    pltpu.sync_copy(idx_hbm, idx_v)
    base_row = wid * RPT

    def scan_body(k, _):
        x = idx_v[pl.ds(k * L, L)]
        jv = k * L + lanes
        local = x - base_row
        valid = (local >= 0) & (local < RPT)
        comp = jnp.where(valid, (local << (JBITS + 1)) | jv, INVALID)
        comp_s, _unused = plsc.sort_key_val(comp, comp)
        loc_s = lax.shift_right_arithmetic(comp_s, JBITS + 1)
        j_s = comp_s & (B - 1)
        valid_s = comp_s < (1 << (JBITS + 16))
        nxt = comp_s.at[shift].get(mode="promise_in_bounds")
        nxt_loc = lax.shift_right_arithmetic(nxt, JBITS + 1)
        win = valid_s & ((loc_s != nxt_loc) | (lanes == L - 1))
        loc_c = jnp.minimum(loc_s, RPT - 1)
        plsc.store_scatter(tag_v, [loc_c], j_s, mask=win)
        return _

    lax.fori_loop(0, B // L, scan_body, None)
    pltpu.sync_copy(tag_v, tag_hbm.at[pl.ds(wid * RPT, RPT)])


def _out_body(mem_hbm, idx_hbm, val_hbm, samp_hbm, tag_hbm, out_hbm,
              idx_v, samp_v, t_v, tc_v, mf_v, g_v, v_v, gsem, vsem):
    cid = lax.axis_index("c")
    sid = lax.axis_index("s")

    pltpu.sync_copy(idx_hbm, idx_v)
    base_s = (cid * NS + sid) * SPW
    pltpu.sync_copy(samp_hbm.at[pl.ds(base_s, SPW)], samp_v)

    pltpu.async_copy(tag_hbm.at[samp_v], t_v, gsem).wait()

    def mask_body(i, _):
        t = t_v[pl.ds(i * L, L)]
        s = samp_v[pl.ds(i * L, L)]
        inb = (t >= 0) & (t < B)
        tc = jnp.where(inb, t, 0)
        back = plsc.load_gather(idx_v, [tc])
        hit = inb & (back == s)
        tc_v[pl.ds(i * L, L)] = tc
        mf_v[pl.ds(i * L, L)] = jnp.where(hit, 1.0, 0.0).astype(jnp.float32)
        return _

    lax.fori_loop(0, SPW // L, mask_body, None)
    gd = pltpu.async_copy(mem_hbm.at[samp_v], g_v, gsem)
    vd = pltpu.async_copy(val_hbm.at[tc_v], v_v, vsem)
    gd.wait()
    vd.wait()

    def row_body(r, _):
        mrow = plsc.load_gather(mf_v, [jnp.full((L,), r, jnp.int32)])
        for cc in range(D // L):
            g = g_v[r, pl.ds(cc * L, L)]
            v = v_v[r, pl.ds(cc * L, L)]
            g_v[r, pl.ds(cc * L, L)] = g + mrow * (v - g)
        return _

    lax.fori_loop(0, SPW, row_body, None)
    pltpu.sync_copy(g_v, out_hbm.at[pl.ds(base_s, SPW)])


@jax.jit
def kernel(mem, idx, val, sample_idx):
    mesh = plsc.VectorSubcoreMesh(
        core_axis_name="c", subcore_axis_name="s",
        num_cores=NC, num_subcores=NS)
    params = pltpu.CompilerParams(
        needs_layout_passes=False, use_tc_tiling_on_sc=False)
    tag_run = pl.kernel(
        _tag_body,
        out_type=jax.ShapeDtypeStruct((MPAD,), jnp.int32),
        mesh=mesh,
        scratch_types=[
            pltpu.VMEM((B,), jnp.int32),
            pltpu.VMEM((RPT,), jnp.int32),
        ],
        compiler_params=params,
    )
    out_run = pl.kernel(
        _out_body,
        out_type=jax.ShapeDtypeStruct((B, D), jnp.float32),
        mesh=mesh,
        scratch_types=[
            pltpu.VMEM((B,), jnp.int32),
            pltpu.VMEM((SPW,), jnp.int32),
            pltpu.VMEM((SPW,), jnp.int32),
            pltpu.VMEM((SPW,), jnp.int32),
            pltpu.VMEM((SPW,), jnp.float32),
            pltpu.VMEM((SPW, D), jnp.float32),
            pltpu.VMEM((SPW, D), jnp.float32),
            pltpu.SemaphoreType.DMA,
            pltpu.SemaphoreType.DMA,
        ],
        compiler_params=params,
    )
    tag = tag_run(idx)
    return out_run(mem, idx, val, sample_idx, tag)


# speedup vs baseline: 1.0333x; 1.0333x over previous
import jax
import jax.numpy as jnp
from jax import lax
from jax.experimental import pallas as pl
from jax.experimental.pallas import tpu as pltpu
from jax.experimental.pallas import tpu_sc as plsc

M = 1000000
B = 16384
D = 64
L = 16
NC = 2
NS = 16
MPAD = 1 << 20
RPT = MPAD // NS
JBITS = 14
SPW = B // (NC * NS)
INVALID = 0x7FFFFFFF


def _join_body(idx_hbm, val_hbm, samp_hbm, vp_hbm, mh_hbm,
               idx_v, samp_v, tag_v, t_v, tc_v, mf_v, v_v, tag_hbm):
    cid = lax.axis_index("c")
    sid = lax.axis_index("s")
    lanes = lax.iota(jnp.int32, L)
    shift = jnp.minimum(lanes + 1, L - 1)

    pltpu.sync_copy(idx_hbm, idx_v)
    base_row = sid * RPT

    def scan_body(k, _):
        x = idx_v[pl.ds(k * L, L)]
        jv = k * L + lanes
        local = x - base_row
        valid = (local >= 0) & (local < RPT)
        comp = jnp.where(valid, (local << JBITS) | jv, INVALID)
        comp_s, _unused = plsc.sort_key_val(comp, comp)
        loc_s = lax.shift_right_arithmetic(comp_s, JBITS)
        j_s = comp_s & (B - 1)
        valid_s = comp_s < (1 << (JBITS + 16))
        nxt = comp_s.at[shift].get(mode="promise_in_bounds")
        nxt_loc = lax.shift_right_arithmetic(nxt, JBITS)
        win = valid_s & ((loc_s != nxt_loc) | (lanes == L - 1))
        loc_c = jnp.minimum(loc_s, RPT - 1)
        plsc.store_scatter(tag_v, [loc_c], j_s, mask=win)
        return _

    lax.fori_loop(0, B // L, scan_body, None)

    pltpu.sync_copy(tag_v, tag_hbm.at[pl.ds(cid * MPAD + sid * RPT, RPT)])
    plsc.subcore_barrier()

    base_s = (cid * NS + sid) * SPW
    pltpu.sync_copy(samp_hbm.at[pl.ds(base_s, SPW)], samp_v)
    tag_half = tag_hbm.at[pl.ds(cid * MPAD, MPAD)]
    pltpu.sync_copy(tag_half.at[samp_v], t_v)

    def mask_body(i, _):
        t = t_v[pl.ds(i * L, L)]
        s = samp_v[pl.ds(i * L, L)]
        inb = (t >= 0) & (t < B)
        tc = jnp.where(inb, t, 0)
        back = plsc.load_gather(idx_v, [tc])
        hit = inb & (back == s)
        tc_v[pl.ds(i * L, L)] = tc
        mf_v[pl.ds(i * L, L)] = jnp.where(hit, 1.0, 0.0).astype(jnp.float32)
        return _

    lax.fori_loop(0, SPW // L, mask_body, None)
    pltpu.sync_copy(val_hbm.at[tc_v], v_v)

    def vmask_body(r, _):
        mrow = plsc.load_gather(mf_v, [jnp.full((L,), r, jnp.int32)])
        for cc in range(D // L):
            v_v[r, pl.ds(cc * L, L)] = v_v[r, pl.ds(cc * L, L)] * mrow
        return _

    lax.fori_loop(0, SPW, vmask_body, None)
    pltpu.sync_copy(v_v, vp_hbm.at[pl.ds(base_s, SPW)])
    pltpu.sync_copy(mf_v, mh_hbm.at[pl.ds(base_s, SPW)])


def _mem_body(mem_hbm, samp_hbm, vp_hbm, mh_hbm, out_hbm,
              samp_v, mh_v, vp_v, g_v, gsem):
    cid = lax.axis_index("c")
    sid = lax.axis_index("s")

    base_s = (cid * NS + sid) * SPW
    pltpu.sync_copy(samp_hbm.at[pl.ds(base_s, SPW)], samp_v)
    gd = pltpu.async_copy(mem_hbm.at[samp_v], g_v, gsem)
    pltpu.sync_copy(vp_hbm.at[pl.ds(base_s, SPW)], vp_v)
    pltpu.sync_copy(mh_hbm.at[pl.ds(base_s, SPW)], mh_v)
    gd.wait()

    def row_body(r, _):
        mrow = plsc.load_gather(mh_v, [jnp.full((L,), r, jnp.int32)])
        for cc in range(D // L):
            g = g_v[r, pl.ds(cc * L, L)]
            vp = vp_v[r, pl.ds(cc * L, L)]
            g_v[r, pl.ds(cc * L, L)] = g - mrow * g + vp
        return _

    lax.fori_loop(0, SPW, row_body, None)
    pltpu.sync_copy(g_v, out_hbm.at[pl.ds(base_s, SPW)])


@jax.jit
def kernel(mem, idx, val, sample_idx):
    mesh = plsc.VectorSubcoreMesh(
        core_axis_name="c", subcore_axis_name="s",
        num_cores=NC, num_subcores=NS)
    params = pltpu.CompilerParams(
        needs_layout_passes=False, use_tc_tiling_on_sc=False)
    join_run = pl.kernel(
        _join_body,
        out_type=(jax.ShapeDtypeStruct((B, D), jnp.float32),
                  jax.ShapeDtypeStruct((B,), jnp.float32)),
        mesh=mesh,
        scratch_types=[
            pltpu.VMEM((B,), jnp.int32),
            pltpu.VMEM((SPW,), jnp.int32),
            pltpu.VMEM((RPT,), jnp.int32),
            pltpu.VMEM((SPW,), jnp.int32),
            pltpu.VMEM((SPW,), jnp.int32),
            pltpu.VMEM((SPW,), jnp.float32),
            pltpu.VMEM((SPW, D), jnp.float32),
            pltpu.HBM((NC * MPAD,), jnp.int32),
        ],
        compiler_params=params,
    )
    mem_run = pl.kernel(
        _mem_body,
        out_type=jax.ShapeDtypeStruct((B, D), jnp.float32),
        mesh=mesh,
        scratch_types=[
            pltpu.VMEM((SPW,), jnp.int32),
            pltpu.VMEM((SPW,), jnp.float32),
            pltpu.VMEM((SPW, D), jnp.float32),
            pltpu.VMEM((SPW, D), jnp.float32),
            pltpu.SemaphoreType.DMA,
        ],
        compiler_params=params,
    )
    vp, mh = join_run(idx, val, sample_idx)
    return mem_run(mem, sample_idx, vp, mh)
